# baseline (speedup 1.0000x reference)
import functools

import jax
import jax.numpy as jnp
from jax import lax
from jax.experimental import pallas as pl
from jax.experimental.pallas import tpu as pltpu
from jax.experimental.pallas import tpu_sc as plsc

ROWS = 4096
COLS = 200
DIM = 4
NUM_EMB = 8

_info = plsc.get_sparse_core_info()
NC = _info.num_cores
NS = _info.num_subcores
NW = NC * NS
IBLK = ROWS // NW
JBLK = 8
NJB = COLS // JBLK


def _make_lookup():
    mesh = plsc.VectorSubcoreMesh(core_axis_name="c", subcore_axis_name="s")

    @functools.partial(
        pl.kernel,
        mesh=mesh,
        compiler_params=pltpu.CompilerParams(
            needs_layout_passes=False,
            use_tc_tiling_on_sc=False,
        ),
        out_type=jax.ShapeDtypeStruct((COLS, NW, DIM, IBLK), jnp.float32),
        scratch_types=[
            pltpu.VMEM((DIM * NUM_EMB,), jnp.float32),
            pltpu.VMEM((NJB, JBLK, IBLK), jnp.int32),
            pltpu.VMEM((COLS, DIM, IBLK), jnp.float32),
        ],
    )
    def lookup(ids_hbm, table_hbm, out_hbm, table_v, idx_v, out_v):
        wid = lax.axis_index("s") * NC + lax.axis_index("c")
        pltpu.sync_copy(table_hbm, table_v)
        pltpu.sync_copy(ids_hbm.at[:, wid], idx_v)

        @plsc.parallel_loop(0, NJB, unroll=1)
        def body(jb):
            for jr in range(JBLK):
                for k in range(IBLK // 16):
                    ids16 = idx_v[jb, jr, pl.ds(k * 16, 16)]
                    for d in range(DIM):
                        vals = plsc.load_gather(
                            table_v.at[pl.ds(NUM_EMB * d, NUM_EMB)], [ids16]
                        )
                        out_v[jb * JBLK + jr, d, pl.ds(k * 16, 16)] = vals

        pltpu.sync_copy(out_v, out_hbm.at[:, wid])

    return lookup


_lookup = _make_lookup()


def kernel(input_ids, embed_table):
    ids4 = (
        input_ids.astype(jnp.int32)
        .T.reshape(NJB, JBLK, NW, IBLK)
        .transpose(0, 2, 1, 3)
    )
    b = _lookup(ids4, embed_table.T.reshape(-1))
    return b.transpose(1, 3, 0, 2).reshape(ROWS, COLS, DIM)
